# Initial kernel scaffold; baseline (speedup 1.0000x reference)
import jax
import jax.numpy as jnp
from jax.experimental import pallas as pl


def kernel(x, edge_index, W1, b1, W2, b2):
    raise NotImplementedError("write your pallas kernel here")


# speedup vs baseline: 7.1092x; 7.1092x over previous
import jax
import jax.numpy as jnp
from jax import lax
from jax.experimental import pallas as pl
from jax.experimental.pallas import tpu as pltpu
from jax.experimental.pallas import tpu_sc as plsc

N = 10000
D = 128
DH = D // 2
E = 320000
NC = 2
NS = 16
EPT = E // NS
C = 80
NCHUNK = EPT // C
NP = 10240
RPT = NP // NS
DEG_L = 16
RB = 2000
GRID = N // RB

_MESH = dict(core_axis_name="c", subcore_axis_name="s",
             num_cores=NC, num_subcores=NS)


def _deg_body(e_hbm, ones_hbm, zeros_hbm, out_hbm,
              idx_v, ones_v, acc):
    c = lax.axis_index("c")
    s = lax.axis_index("s")
    pltpu.sync_copy(e_hbm.at[c, s], idx_v)
    pltpu.sync_copy(ones_hbm, ones_v)
    row = pl.ds(s * RPT, RPT)
    pltpu.sync_copy(zeros_hbm, acc.at[row])
    plsc.subcore_barrier()

    def body(j, carry):
        pltpu.sync_copy(ones_v, acc.at[idx_v.at[j]], add=True)
        return carry

    lax.fori_loop(0, NCHUNK, body, 0)
    plsc.subcore_barrier()
    pltpu.sync_copy(acc.at[row], out_hbm.at[c, row])


def _deg_call(e2, ones16, zeros16):
    fn = pl.kernel(
        _deg_body,
        out_type=jax.ShapeDtypeStruct((NC, NP, DEG_L), jnp.float32),
        mesh=plsc.VectorSubcoreMesh(**_MESH),
        compiler_params=pltpu.CompilerParams(use_tc_tiling_on_sc=False),
        scratch_types=[
            pltpu.VMEM((NCHUNK, C), jnp.int32),
            pltpu.VMEM((C, DEG_L), jnp.float32),
            pltpu.VMEM_SHARED((NP, DEG_L), jnp.float32),
        ],
    )
    return fn(e2, ones16, zeros16)


def _agg_body(th_hbm, snd_hbm, rcv_hbm, zeros_hbm, out_hbm,
              snd_v, rcv_v, rb0, rb1, acc, sem0, sem1):
    c = lax.axis_index("c")
    s = lax.axis_index("s")
    pltpu.sync_copy(snd_hbm.at[s], snd_v)
    pltpu.sync_copy(rcv_hbm.at[s], rcv_v)
    row = pl.ds(s * RPT, RPT)
    pltpu.sync_copy(zeros_hbm, acc.at[row])
    plsc.subcore_barrier()
    tc_ref = th_hbm.at[c]

    def body(jj, carry):
        j0 = jj * 2
        j1 = j0 + 1
        g0 = pltpu.async_copy(tc_ref.at[snd_v.at[j0]], rb0, sem0)
        g1 = pltpu.async_copy(tc_ref.at[snd_v.at[j1]], rb1, sem1)
        g0.wait()
        pltpu.sync_copy(rb0, acc.at[rcv_v.at[j0]], add=True)
        g1.wait()
        pltpu.sync_copy(rb1, acc.at[rcv_v.at[j1]], add=True)
        return carry

    lax.fori_loop(0, NCHUNK // 2, body, 0)
    plsc.subcore_barrier()
    pltpu.sync_copy(acc.at[row], out_hbm.at[c, row])


def _agg_call(th, snd, rcv, zerosH):
    fn = pl.kernel(
        _agg_body,
        out_type=jax.ShapeDtypeStruct((NC, NP, DH), jnp.float32),
        mesh=plsc.VectorSubcoreMesh(**_MESH),
        compiler_params=pltpu.CompilerParams(use_tc_tiling_on_sc=False),
        scratch_types=[
            pltpu.VMEM((NCHUNK, C), jnp.int32),
            pltpu.VMEM((NCHUNK, C), jnp.int32),
            pltpu.VMEM((C, DH), jnp.float32),
            pltpu.VMEM((C, DH), jnp.float32),
            pltpu.VMEM_SHARED((NP, DH), jnp.float32),
            pltpu.SemaphoreType.DMA,
            pltpu.SemaphoreType.DMA,
        ],
    )
    return fn(th, snd, rcv, zerosH)


def _l1_body(x_ref, w_ref, b_ref, dp_ref, o_ref):
    ns = lax.rsqrt(jnp.maximum(dp_ref[0, 0], 1.0))
    h = jnp.dot(x_ref[...], w_ref[...], preferred_element_type=jnp.float32)
    h = (h + b_ref[...][None, :]) * ns[:, None]
    o_ref[0] = h[:, :DH]
    o_ref[1] = h[:, DH:]


def _l1_call(x, W1, b1, dp):
    return pl.pallas_call(
        _l1_body,
        grid=(GRID,),
        in_specs=[
            pl.BlockSpec((RB, D), lambda i: (i, 0)),
            pl.BlockSpec((D, D), lambda i: (0, 0)),
            pl.BlockSpec((D,), lambda i: (0,)),
            pl.BlockSpec((1, NC, RB), lambda i: (i, 0, 0)),
        ],
        out_specs=pl.BlockSpec((NC, RB, DH), lambda i: (0, i, 0)),
        out_shape=jax.ShapeDtypeStruct((NC, N, DH), jnp.float32),
    )(x, W1, b1, dp)


def _l2_body(p_ref, w_ref, b_ref, dp_ref, o_ref):
    nr = lax.rsqrt(jnp.maximum(dp_ref[0, 1], 1.0))
    ns = lax.rsqrt(jnp.maximum(dp_ref[0, 0], 1.0))
    a = jnp.concatenate([p_ref[0], p_ref[1]], axis=-1)
    h1 = jnp.maximum(a * nr[:, None], 0.0)
    h = jnp.dot(h1, w_ref[...], preferred_element_type=jnp.float32)
    h = (h + b_ref[...][None, :]) * ns[:, None]
    o_ref[0] = h[:, :DH]
    o_ref[1] = h[:, DH:]


def _l2_call(p, W2, b2, dp):
    return pl.pallas_call(
        _l2_body,
        grid=(GRID,),
        in_specs=[
            pl.BlockSpec((NC, RB, DH), lambda i: (0, i, 0)),
            pl.BlockSpec((D, D), lambda i: (0, 0)),
            pl.BlockSpec((D,), lambda i: (0,)),
            pl.BlockSpec((1, NC, RB), lambda i: (i, 0, 0)),
        ],
        out_specs=pl.BlockSpec((NC, RB, DH), lambda i: (0, i, 0)),
        out_shape=jax.ShapeDtypeStruct((NC, N, DH), jnp.float32),
    )(p, W2, b2, dp)


def _out_body(p_ref, dp_ref, o_ref):
    nr = lax.rsqrt(jnp.maximum(dp_ref[0, 1], 1.0))
    a = jnp.concatenate([p_ref[0], p_ref[1]], axis=-1)
    o_ref[...] = jnp.maximum(a * nr[:, None], 0.0)


def _out_call(p, dp):
    return pl.pallas_call(
        _out_body,
        grid=(GRID,),
        in_specs=[
            pl.BlockSpec((NC, RB, DH), lambda i: (0, i, 0)),
            pl.BlockSpec((1, NC, RB), lambda i: (i, 0, 0)),
        ],
        out_specs=pl.BlockSpec((RB, D), lambda i: (i, 0)),
        out_shape=jax.ShapeDtypeStruct((N, D), jnp.float32),
    )(p, dp)


def kernel(x, edge_index, W1, b1, W2, b2):
    e2 = edge_index.reshape(2, NS, NCHUNK, C)
    snd = e2[0]
    rcv = e2[1]
    ones16 = jnp.ones((C, DEG_L), jnp.float32)
    zeros16 = jnp.zeros((RPT, DEG_L), jnp.float32)
    zerosH = jnp.zeros((RPT, DH), jnp.float32)

    degp = _deg_call(e2, ones16, zeros16)
    dp = degp[:, :N, 0].reshape(NC, GRID, RB).transpose(1, 0, 2)
    t1 = _l1_call(x, W1, b1, dp)
    p1 = _agg_call(t1, snd, rcv, zerosH)
    t2 = _l2_call(p1, W2, b2, dp)
    p2 = _agg_call(t2, snd, rcv, zerosH)
    return _out_call(p2, dp)


# speedup vs baseline: 10.3563x; 1.4567x over previous
import jax
import jax.numpy as jnp
from jax import lax
from jax.experimental import pallas as pl
from jax.experimental.pallas import tpu as pltpu
from jax.experimental.pallas import tpu_sc as plsc

N = 10000
D = 128
DH = D // 2
E = 320000
NC = 2
NS = 16
EPT = E // NS
C = 80
NCHUNK = EPT // C
NP = 10240
RPT = NP // NS
DEG_L = 16
RB = 2000
GRID = N // RB

_MESH = dict(core_axis_name="c", subcore_axis_name="s",
             num_cores=NC, num_subcores=NS)


def _deg_body(e_hbm, ones_hbm, zeros_hbm, out_hbm,
              idx_v, ones_v, acc):
    c = lax.axis_index("c")
    s = lax.axis_index("s")
    pltpu.sync_copy(e_hbm.at[c, s], idx_v)
    pltpu.sync_copy(ones_hbm, ones_v)
    row = pl.ds(s * RPT, RPT)
    pltpu.sync_copy(zeros_hbm, acc.at[row])
    plsc.subcore_barrier()

    def body(j, carry):
        pltpu.sync_copy(ones_v, acc.at[idx_v.at[j]], add=True)
        return carry

    lax.fori_loop(0, NCHUNK, body, 0)
    plsc.subcore_barrier()
    pltpu.sync_copy(acc.at[row], out_hbm.at[c, row])


def _deg_call(e2, ones16, zeros16):
    fn = pl.kernel(
        _deg_body,
        out_type=jax.ShapeDtypeStruct((NC, NP, DEG_L), jnp.float32),
        mesh=plsc.VectorSubcoreMesh(**_MESH),
        compiler_params=pltpu.CompilerParams(use_tc_tiling_on_sc=False),
        scratch_types=[
            pltpu.VMEM((NCHUNK, C), jnp.int32),
            pltpu.VMEM((C, DEG_L), jnp.float32),
            pltpu.VMEM_SHARED((NP, DEG_L), jnp.float32),
        ],
    )
    return fn(e2, ones16, zeros16)


NB = 5
NROUND = NCHUNK // NB


def _agg_body(th_hbm, snd_hbm, rcv_hbm, zeros_hbm, out_hbm,
              snd_v, rcv_v, rbs, gsems, ssems, acc):
    c = lax.axis_index("c")
    s = lax.axis_index("s")
    pltpu.sync_copy(snd_hbm.at[s], snd_v)
    pltpu.sync_copy(rcv_hbm.at[s], rcv_v)
    row = pl.ds(s * RPT, RPT)
    pltpu.sync_copy(zeros_hbm, acc.at[row])
    plsc.subcore_barrier()
    tc_ref = th_hbm.at[c]

    for b in range(NB):
        pltpu.async_copy(tc_ref.at[snd_v.at[b]], rbs.at[b], gsems.at[b])

    def body(jj, carry):
        base = jj * NB
        for b in range(NB):
            j = base + b
            pltpu.make_async_copy(tc_ref.at[snd_v.at[j]], rbs.at[b],
                                  gsems.at[b]).wait()
            pltpu.async_copy(rbs.at[b], acc.at[rcv_v.at[j]], ssems.at[b],
                             add=True)
        for b in range(NB):
            j = base + b
            pltpu.make_async_copy(rbs.at[b], acc.at[rcv_v.at[j]],
                                  ssems.at[b]).wait()

            @pl.when(jj + 1 < NROUND)
            def _():
                pltpu.async_copy(tc_ref.at[snd_v.at[j + NB]], rbs.at[b],
                                 gsems.at[b])

        return carry

    lax.fori_loop(0, NROUND, body, 0)
    plsc.subcore_barrier()
    pltpu.sync_copy(acc.at[row], out_hbm.at[c, row])


def _agg_call(th, snd, rcv, zerosH):
    fn = pl.kernel(
        _agg_body,
        out_type=jax.ShapeDtypeStruct((NC, NP, DH), jnp.float32),
        mesh=plsc.VectorSubcoreMesh(**_MESH),
        compiler_params=pltpu.CompilerParams(use_tc_tiling_on_sc=False),
        scratch_types=[
            pltpu.VMEM((NCHUNK, C), jnp.int32),
            pltpu.VMEM((NCHUNK, C), jnp.int32),
            pltpu.VMEM((NB, C, DH), jnp.float32),
            pltpu.SemaphoreType.DMA((NB,)),
            pltpu.SemaphoreType.DMA((NB,)),
            pltpu.VMEM_SHARED((NP, DH), jnp.float32),
        ],
    )
    return fn(th, snd, rcv, zerosH)


def _l1_body(x_ref, w_ref, b_ref, dp_ref, o_ref):
    ns = lax.rsqrt(jnp.maximum(dp_ref[0, 0], 1.0))
    h = jnp.dot(x_ref[...], w_ref[...], preferred_element_type=jnp.float32)
    h = (h + b_ref[...][None, :]) * ns[:, None]
    o_ref[0] = h[:, :DH]
    o_ref[1] = h[:, DH:]


def _l1_call(x, W1, b1, dp):
    return pl.pallas_call(
        _l1_body,
        grid=(GRID,),
        in_specs=[
            pl.BlockSpec((RB, D), lambda i: (i, 0)),
            pl.BlockSpec((D, D), lambda i: (0, 0)),
            pl.BlockSpec((D,), lambda i: (0,)),
            pl.BlockSpec((1, NC, RB), lambda i: (i, 0, 0)),
        ],
        out_specs=pl.BlockSpec((NC, RB, DH), lambda i: (0, i, 0)),
        out_shape=jax.ShapeDtypeStruct((NC, N, DH), jnp.float32),
    )(x, W1, b1, dp)


def _l2_body(p_ref, w_ref, b_ref, dp_ref, o_ref):
    nr = lax.rsqrt(jnp.maximum(dp_ref[0, 1], 1.0))
    ns = lax.rsqrt(jnp.maximum(dp_ref[0, 0], 1.0))
    a = jnp.concatenate([p_ref[0], p_ref[1]], axis=-1)
    h1 = jnp.maximum(a * nr[:, None], 0.0)
    h = jnp.dot(h1, w_ref[...], preferred_element_type=jnp.float32)
    h = (h + b_ref[...][None, :]) * ns[:, None]
    o_ref[0] = h[:, :DH]
    o_ref[1] = h[:, DH:]


def _l2_call(p, W2, b2, dp):
    return pl.pallas_call(
        _l2_body,
        grid=(GRID,),
        in_specs=[
            pl.BlockSpec((NC, RB, DH), lambda i: (0, i, 0)),
            pl.BlockSpec((D, D), lambda i: (0, 0)),
            pl.BlockSpec((D,), lambda i: (0,)),
            pl.BlockSpec((1, NC, RB), lambda i: (i, 0, 0)),
        ],
        out_specs=pl.BlockSpec((NC, RB, DH), lambda i: (0, i, 0)),
        out_shape=jax.ShapeDtypeStruct((NC, N, DH), jnp.float32),
    )(p, W2, b2, dp)


def _out_body(p_ref, dp_ref, o_ref):
    nr = lax.rsqrt(jnp.maximum(dp_ref[0, 1], 1.0))
    a = jnp.concatenate([p_ref[0], p_ref[1]], axis=-1)
    o_ref[...] = jnp.maximum(a * nr[:, None], 0.0)


def _out_call(p, dp):
    return pl.pallas_call(
        _out_body,
        grid=(GRID,),
        in_specs=[
            pl.BlockSpec((NC, RB, DH), lambda i: (0, i, 0)),
            pl.BlockSpec((1, NC, RB), lambda i: (i, 0, 0)),
        ],
        out_specs=pl.BlockSpec((RB, D), lambda i: (i, 0)),
        out_shape=jax.ShapeDtypeStruct((N, D), jnp.float32),
    )(p, dp)


def kernel(x, edge_index, W1, b1, W2, b2):
    e2 = edge_index.reshape(2, NS, NCHUNK, C)
    snd = e2[0]
    rcv = e2[1]
    ones16 = jnp.ones((C, DEG_L), jnp.float32)
    zeros16 = jnp.zeros((RPT, DEG_L), jnp.float32)
    zerosH = jnp.zeros((RPT, DH), jnp.float32)

    degp = _deg_call(e2, ones16, zeros16)
    dp = degp[:, :N, 0].reshape(NC, GRID, RB).transpose(1, 0, 2)
    t1 = _l1_call(x, W1, b1, dp)
    p1 = _agg_call(t1, snd, rcv, zerosH)
    t2 = _l2_call(p1, W2, b2, dp)
    p2 = _agg_call(t2, snd, rcv, zerosH)
    return _out_call(p2, dp)


# speedup vs baseline: 10.7095x; 1.0341x over previous
import jax
import jax.numpy as jnp
from jax import lax
from jax.experimental import pallas as pl
from jax.experimental.pallas import tpu as pltpu
from jax.experimental.pallas import tpu_sc as plsc

N = 10000
D = 128
DH = D // 2
E = 320000
NC = 2
NS = 16
EPT = E // NS
C = 80
NCHUNK = EPT // C
NP = 10240
RPT = NP // NS
DEG_L = 16
RB = 2000
GRID = N // RB

_MESH = dict(core_axis_name="c", subcore_axis_name="s",
             num_cores=NC, num_subcores=NS)


DNB = 10


def _deg_body(e_hbm, ones_hbm, zeros_hbm, out_hbm,
              idx_v, ones_v, dsems, acc):
    c = lax.axis_index("c")
    s = lax.axis_index("s")
    pltpu.sync_copy(e_hbm.at[c, s], idx_v)
    pltpu.sync_copy(ones_hbm, ones_v)
    row = pl.ds(s * RPT, RPT)
    pltpu.sync_copy(zeros_hbm, acc.at[row])
    plsc.subcore_barrier()

    def body(jj, carry):
        base = jj * DNB
        for b in range(DNB):
            pltpu.async_copy(ones_v, acc.at[idx_v.at[base + b]], dsems.at[b],
                             add=True)
        for b in range(DNB):
            pltpu.make_async_copy(ones_v, acc.at[idx_v.at[base + b]],
                                  dsems.at[b]).wait()
        return carry

    lax.fori_loop(0, NCHUNK // DNB, body, 0)
    plsc.subcore_barrier()
    pltpu.sync_copy(acc.at[row], out_hbm.at[c, row])


def _deg_call(e2, ones16, zeros16):
    fn = pl.kernel(
        _deg_body,
        out_type=jax.ShapeDtypeStruct((NC, NP, DEG_L), jnp.float32),
        mesh=plsc.VectorSubcoreMesh(**_MESH),
        compiler_params=pltpu.CompilerParams(use_tc_tiling_on_sc=False),
        scratch_types=[
            pltpu.VMEM((NCHUNK, C), jnp.int32),
            pltpu.VMEM((C, DEG_L), jnp.float32),
            pltpu.SemaphoreType.DMA((DNB,)),
            pltpu.VMEM_SHARED((NP, DEG_L), jnp.float32),
        ],
    )
    return fn(e2, ones16, zeros16)


NB = 5
NROUND = NCHUNK // NB


def _agg_body(th_hbm, snd_hbm, rcv_hbm, zeros_hbm, out_hbm,
              snd_v, rcv_v, rbs, gsems, ssems, acc):
    c = lax.axis_index("c")
    s = lax.axis_index("s")
    pltpu.sync_copy(snd_hbm.at[s], snd_v)
    pltpu.sync_copy(rcv_hbm.at[s], rcv_v)
    row = pl.ds(s * RPT, RPT)
    pltpu.sync_copy(zeros_hbm, acc.at[row])
    plsc.subcore_barrier()
    tc_ref = th_hbm.at[c]

    for b in range(NB):
        pltpu.async_copy(tc_ref.at[snd_v.at[b]], rbs.at[b], gsems.at[b])

    def body(jj, carry):
        base = jj * NB
        for b in range(NB):
            j = base + b
            pltpu.make_async_copy(tc_ref.at[snd_v.at[j]], rbs.at[b],
                                  gsems.at[b]).wait()
            pltpu.async_copy(rbs.at[b], acc.at[rcv_v.at[j]], ssems.at[b],
                             add=True)
        for b in range(NB):
            j = base + b
            pltpu.make_async_copy(rbs.at[b], acc.at[rcv_v.at[j]],
                                  ssems.at[b]).wait()

            @pl.when(jj + 1 < NROUND)
            def _():
                pltpu.async_copy(tc_ref.at[snd_v.at[j + NB]], rbs.at[b],
                                 gsems.at[b])

        return carry

    lax.fori_loop(0, NROUND, body, 0)
    plsc.subcore_barrier()
    pltpu.sync_copy(acc.at[row], out_hbm.at[c, row])


def _agg_call(th, snd, rcv, zerosH):
    fn = pl.kernel(
        _agg_body,
        out_type=jax.ShapeDtypeStruct((NC, NP, DH), jnp.float32),
        mesh=plsc.VectorSubcoreMesh(**_MESH),
        compiler_params=pltpu.CompilerParams(use_tc_tiling_on_sc=False),
        scratch_types=[
            pltpu.VMEM((NCHUNK, C), jnp.int32),
            pltpu.VMEM((NCHUNK, C), jnp.int32),
            pltpu.VMEM((NB, C, DH), jnp.float32),
            pltpu.SemaphoreType.DMA((NB,)),
            pltpu.SemaphoreType.DMA((NB,)),
            pltpu.VMEM_SHARED((NP, DH), jnp.float32),
        ],
    )
    return fn(th, snd, rcv, zerosH)


def _l1_body(x_ref, w_ref, b_ref, dp_ref, o_ref):
    ns = lax.rsqrt(jnp.maximum(dp_ref[0, :N], 1.0))
    h = jnp.dot(x_ref[...], w_ref[...], preferred_element_type=jnp.float32)
    h = (h + b_ref[...][None, :]) * ns[:, None]
    o_ref[0] = h[:, :DH]
    o_ref[1] = h[:, DH:]


def _l1_call(x, W1, b1, dp):
    return pl.pallas_call(
        _l1_body,
        out_shape=jax.ShapeDtypeStruct((NC, N, DH), jnp.float32),
    )(x, W1, b1, dp)


def _l2_body(p_ref, w_ref, b_ref, dp_ref, o_ref):
    nr = lax.rsqrt(jnp.maximum(dp_ref[1, :N], 1.0))
    ns = lax.rsqrt(jnp.maximum(dp_ref[0, :N], 1.0))
    a = jnp.concatenate([p_ref[0, :N], p_ref[1, :N]], axis=-1)
    h1 = jnp.maximum(a * nr[:, None], 0.0)
    h = jnp.dot(h1, w_ref[...], preferred_element_type=jnp.float32)
    h = (h + b_ref[...][None, :]) * ns[:, None]
    o_ref[0] = h[:, :DH]
    o_ref[1] = h[:, DH:]


def _l2_call(p, W2, b2, dp):
    return pl.pallas_call(
        _l2_body,
        out_shape=jax.ShapeDtypeStruct((NC, N, DH), jnp.float32),
    )(p, W2, b2, dp)


def _out_body(p_ref, dp_ref, o_ref):
    nr = lax.rsqrt(jnp.maximum(dp_ref[1, :N], 1.0))
    a = jnp.concatenate([p_ref[0, :N], p_ref[1, :N]], axis=-1)
    o_ref[...] = jnp.maximum(a * nr[:, None], 0.0)


def _out_call(p, dp):
    return pl.pallas_call(
        _out_body,
        out_shape=jax.ShapeDtypeStruct((N, D), jnp.float32),
    )(p, dp)


def kernel(x, edge_index, W1, b1, W2, b2):
    e2 = edge_index.reshape(2, NS, NCHUNK, C)
    snd = e2[0]
    rcv = e2[1]
    ones16 = jnp.ones((C, DEG_L), jnp.float32)
    zeros16 = jnp.zeros((RPT, DEG_L), jnp.float32)
    zerosH = jnp.zeros((RPT, DH), jnp.float32)

    degp = _deg_call(e2, ones16, zeros16)
    dp = degp[..., 0]
    t1 = _l1_call(x, W1, b1, dp)
    p1 = _agg_call(t1, snd, rcv, zerosH)
    t2 = _l2_call(p1, W2, b2, dp)
    p2 = _agg_call(t2, snd, rcv, zerosH)
    return _out_call(p2, dp)


# speedup vs baseline: 10.7366x; 1.0025x over previous
import jax
import jax.numpy as jnp
from jax import lax
from jax.experimental import pallas as pl
from jax.experimental.pallas import tpu as pltpu
from jax.experimental.pallas import tpu_sc as plsc

N = 10000
D = 128
DH = D // 2
E = 320000
NC = 2
NS = 16
EPT = E // NS
C = 80
NCHUNK = EPT // C
NP = 10240
RPT = NP // NS
DEG_L = 16
RB = 2000
GRID = N // RB

_MESH = dict(core_axis_name="c", subcore_axis_name="s",
             num_cores=NC, num_subcores=NS)


DNB = 10


def _deg_body(e_hbm, ones_hbm, zeros_hbm, out_hbm,
              idx_v, ones_v, dsems, acc):
    c = lax.axis_index("c")
    s = lax.axis_index("s")
    pltpu.sync_copy(e_hbm.at[c, s], idx_v)
    pltpu.sync_copy(ones_hbm, ones_v)
    row = pl.ds(s * RPT, RPT)
    pltpu.sync_copy(zeros_hbm, acc.at[row])
    plsc.subcore_barrier()

    def body(jj, carry):
        base = jj * DNB
        for b in range(DNB):
            pltpu.async_copy(ones_v, acc.at[idx_v.at[base + b]], dsems.at[b],
                             add=True)
        for b in range(DNB):
            pltpu.make_async_copy(ones_v, acc.at[idx_v.at[base + b]],
                                  dsems.at[b]).wait()
        return carry

    lax.fori_loop(0, NCHUNK // DNB, body, 0)
    plsc.subcore_barrier()
    pltpu.sync_copy(acc.at[row], out_hbm.at[c, row])


def _deg_call(e2, ones16, zeros16):
    fn = pl.kernel(
        _deg_body,
        out_type=jax.ShapeDtypeStruct((NC, NP, DEG_L), jnp.float32),
        mesh=plsc.VectorSubcoreMesh(**_MESH),
        compiler_params=pltpu.CompilerParams(use_tc_tiling_on_sc=False),
        scratch_types=[
            pltpu.VMEM((NCHUNK, C), jnp.int32),
            pltpu.VMEM((C, DEG_L), jnp.float32),
            pltpu.SemaphoreType.DMA((DNB,)),
            pltpu.VMEM_SHARED((NP, DEG_L), jnp.float32),
        ],
    )
    return fn(e2, ones16, zeros16)


NB = 5
NROUND = NCHUNK // NB


def _agg_body(th_hbm, snd_hbm, rcv_hbm, zeros_hbm, out_hbm,
              snd_v, rcv_v, rbs, gsems, ssems, acc):
    c = lax.axis_index("c")
    s = lax.axis_index("s")
    pltpu.sync_copy(snd_hbm.at[s], snd_v)
    pltpu.sync_copy(rcv_hbm.at[s], rcv_v)
    row = pl.ds(s * RPT, RPT)
    pltpu.sync_copy(zeros_hbm, acc.at[row])
    plsc.subcore_barrier()
    tc_ref = th_hbm.at[c]

    for b in range(NB):
        pltpu.async_copy(tc_ref.at[snd_v.at[b]], rbs.at[b], gsems.at[b])

    def body(jj, carry):
        base = jj * NB
        for b in range(NB):
            j = base + b
            pltpu.make_async_copy(tc_ref.at[snd_v.at[j]], rbs.at[b],
                                  gsems.at[b]).wait()
            pltpu.async_copy(rbs.at[b], acc.at[rcv_v.at[j]], ssems.at[b],
                             add=True)
        for b in range(NB):
            j = base + b
            pltpu.make_async_copy(rbs.at[b], acc.at[rcv_v.at[j]],
                                  ssems.at[b]).wait()

            @pl.when(jj + 1 < NROUND)
            def _():
                pltpu.async_copy(tc_ref.at[snd_v.at[j + NB]], rbs.at[b],
                                 gsems.at[b])

        return carry

    lax.fori_loop(0, NROUND, body, 0)
    plsc.subcore_barrier()
    pltpu.sync_copy(acc.at[row], out_hbm.at[c, row])


EB = 128


def _aggf_body(th_hbm, snd_hbm, rcv_hbm, zeros_hbm, nrb_hbm, out_hbm,
               snd_v, rcv_v, rbs, gsems, ssems, nrb_v, buf_v, acc):
    c = lax.axis_index("c")
    s = lax.axis_index("s")
    pltpu.sync_copy(snd_hbm.at[s], snd_v)
    pltpu.sync_copy(rcv_hbm.at[s], rcv_v)
    row = pl.ds(s * RPT, RPT)
    pltpu.sync_copy(zeros_hbm, acc.at[row])
    plsc.subcore_barrier()
    tc_ref = th_hbm.at[c]

    for b in range(NB):
        pltpu.async_copy(tc_ref.at[snd_v.at[b]], rbs.at[b], gsems.at[b])

    def body(jj, carry):
        base = jj * NB
        for b in range(NB):
            j = base + b
            pltpu.make_async_copy(tc_ref.at[snd_v.at[j]], rbs.at[b],
                                  gsems.at[b]).wait()
            pltpu.async_copy(rbs.at[b], acc.at[rcv_v.at[j]], ssems.at[b],
                             add=True)
        for b in range(NB):
            j = base + b
            pltpu.make_async_copy(rbs.at[b], acc.at[rcv_v.at[j]],
                                  ssems.at[b]).wait()

            @pl.when(jj + 1 < NROUND)
            def _():
                pltpu.async_copy(tc_ref.at[snd_v.at[j + NB]], rbs.at[b],
                                 gsems.at[b])

        return carry

    lax.fori_loop(0, NROUND, body, 0)
    plsc.subcore_barrier()
    def eblk(t, carry):
        r0 = s * RPT + t * EB
        pltpu.sync_copy(acc.at[pl.ds(r0, EB)], buf_v)
        pltpu.sync_copy(nrb_hbm.at[pl.ds(r0, EB)], nrb_v)

        def srow(r, carry2):
            nv = nrb_v[r]
            for k in range(DH // 16):
                col = pl.ds(k * 16, 16)
                buf_v[r, col] = jnp.maximum(buf_v[r, col] * nv, 0.0)
            return carry2

        lax.fori_loop(0, EB, srow, 0)
        pltpu.sync_copy(buf_v, out_hbm.at[pl.ds(r0, EB), pl.ds(c * DH, DH)])
        return carry

    lax.fori_loop(0, RPT // EB, eblk, 0)


def _aggf_call(th, snd, rcv, zerosH, nrb):
    fn = pl.kernel(
        _aggf_body,
        out_type=jax.ShapeDtypeStruct((NP, D), jnp.float32),
        mesh=plsc.VectorSubcoreMesh(**_MESH),
        compiler_params=pltpu.CompilerParams(use_tc_tiling_on_sc=False),
        scratch_types=[
            pltpu.VMEM((NCHUNK, C), jnp.int32),
            pltpu.VMEM((NCHUNK, C), jnp.int32),
            pltpu.VMEM((NB, C, DH), jnp.float32),
            pltpu.SemaphoreType.DMA((NB,)),
            pltpu.SemaphoreType.DMA((NB,)),
            pltpu.VMEM((EB, DEG_L), jnp.float32),
            pltpu.VMEM((EB, DH), jnp.float32),
            pltpu.VMEM_SHARED((NP, DH), jnp.float32),
        ],
    )
    return fn(th, snd, rcv, zerosH, nrb)


def _agg_call(th, snd, rcv, zerosH):
    fn = pl.kernel(
        _agg_body,
        out_type=jax.ShapeDtypeStruct((NC, NP, DH), jnp.float32),
        mesh=plsc.VectorSubcoreMesh(**_MESH),
        compiler_params=pltpu.CompilerParams(use_tc_tiling_on_sc=False),
        scratch_types=[
            pltpu.VMEM((NCHUNK, C), jnp.int32),
            pltpu.VMEM((NCHUNK, C), jnp.int32),
            pltpu.VMEM((NB, C, DH), jnp.float32),
            pltpu.SemaphoreType.DMA((NB,)),
            pltpu.SemaphoreType.DMA((NB,)),
            pltpu.VMEM_SHARED((NP, DH), jnp.float32),
        ],
    )
    return fn(th, snd, rcv, zerosH)


def _l1_body(x_ref, w_ref, b_ref, dp_ref, o_ref):
    ns = lax.rsqrt(jnp.maximum(dp_ref[0, :N], 1.0))
    h = jnp.dot(x_ref[...], w_ref[...], preferred_element_type=jnp.float32)
    h = (h + b_ref[...][None, :]) * ns[:, None]
    o_ref[0] = h[:, :DH]
    o_ref[1] = h[:, DH:]


def _l1_call(x, W1, b1, dp):
    return pl.pallas_call(
        _l1_body,
        out_shape=jax.ShapeDtypeStruct((NC, N, DH), jnp.float32),
    )(x, W1, b1, dp)


def _l2_body(p_ref, w_ref, b_ref, dp_ref, o_ref, onr_ref):
    nr_full = lax.rsqrt(jnp.maximum(dp_ref[1], 1.0))
    nr = nr_full[:N]
    ns = lax.rsqrt(jnp.maximum(dp_ref[0, :N], 1.0))
    a = jnp.concatenate([p_ref[0, :N], p_ref[1, :N]], axis=-1)
    h1 = jnp.maximum(a * nr[:, None], 0.0)
    h = jnp.dot(h1, w_ref[...], preferred_element_type=jnp.float32)
    h = (h + b_ref[...][None, :]) * ns[:, None]
    o_ref[0] = h[:, :DH]
    o_ref[1] = h[:, DH:]
    onr_ref[...] = jnp.broadcast_to(nr_full[:, None], (NP, DEG_L))


def _l2_call(p, W2, b2, dp):
    return pl.pallas_call(
        _l2_body,
        out_shape=[jax.ShapeDtypeStruct((NC, N, DH), jnp.float32),
                   jax.ShapeDtypeStruct((NP, DEG_L), jnp.float32)],
    )(p, W2, b2, dp)


def _out_body(p_ref, dp_ref, o_ref):
    nr = lax.rsqrt(jnp.maximum(dp_ref[1, :N], 1.0))
    a = jnp.concatenate([p_ref[0, :N], p_ref[1, :N]], axis=-1)
    o_ref[...] = jnp.maximum(a * nr[:, None], 0.0)


def _out_call(p, dp):
    return pl.pallas_call(
        _out_body,
        out_shape=jax.ShapeDtypeStruct((N, D), jnp.float32),
    )(p, dp)


def kernel(x, edge_index, W1, b1, W2, b2):
    e2 = edge_index.reshape(2, NS, NCHUNK, C)
    snd = e2[0]
    rcv = e2[1]
    ones16 = jnp.ones((C, DEG_L), jnp.float32)
    zeros16 = jnp.zeros((RPT, DEG_L), jnp.float32)
    zerosH = jnp.zeros((RPT, DH), jnp.float32)

    degp = _deg_call(e2, ones16, zeros16)
    dp = degp[..., 0]
    t1 = _l1_call(x, W1, b1, dp)
    p1 = _agg_call(t1, snd, rcv, zerosH)
    t2, nrb = _l2_call(p1, W2, b2, dp)
    out = _aggf_call(t2, snd, rcv, zerosH, nrb)
    return out[:N]


# speedup vs baseline: 10.7561x; 1.0018x over previous
import jax
import jax.numpy as jnp
from jax import lax
from jax.experimental import pallas as pl
from jax.experimental.pallas import tpu as pltpu
from jax.experimental.pallas import tpu_sc as plsc

N = 10000
D = 128
DH = D // 2
E = 320000
NC = 2
NS = 16
EPT = E // NS
C = 80
NCHUNK = EPT // C
CA = 128
NCHA = 160
EPA = NS * NCHA * CA
NP = 10240
RPT = NP // NS
DEG_L = 16
RB = 2000
GRID = N // RB

_MESH = dict(core_axis_name="c", subcore_axis_name="s",
             num_cores=NC, num_subcores=NS)


DNB = 10


def _deg_body(e_hbm, ones_hbm, zeros_hbm, out_hbm,
              idx_v, ones_v, dsems, acc):
    c = lax.axis_index("c")
    s = lax.axis_index("s")
    pltpu.sync_copy(e_hbm.at[c, s], idx_v)
    pltpu.sync_copy(ones_hbm, ones_v)
    row = pl.ds(s * RPT, RPT)
    pltpu.sync_copy(zeros_hbm, acc.at[row])
    plsc.subcore_barrier()

    def body(jj, carry):
        base = jj * DNB
        for b in range(DNB):
            pltpu.async_copy(ones_v, acc.at[idx_v.at[base + b]], dsems.at[b],
                             add=True)
        for b in range(DNB):
            pltpu.make_async_copy(ones_v, acc.at[idx_v.at[base + b]],
                                  dsems.at[b]).wait()
        return carry

    lax.fori_loop(0, NCHUNK // DNB, body, 0)
    plsc.subcore_barrier()
    pltpu.sync_copy(acc.at[row], out_hbm.at[c, row])


def _deg_call(e2, ones16, zeros16):
    fn = pl.kernel(
        _deg_body,
        out_type=jax.ShapeDtypeStruct((NC, NP, DEG_L), jnp.float32),
        mesh=plsc.VectorSubcoreMesh(**_MESH),
        compiler_params=pltpu.CompilerParams(use_tc_tiling_on_sc=False),
        scratch_types=[
            pltpu.VMEM((NCHUNK, C), jnp.int32),
            pltpu.VMEM((C, DEG_L), jnp.float32),
            pltpu.SemaphoreType.DMA((DNB,)),
            pltpu.VMEM_SHARED((NP, DEG_L), jnp.float32),
        ],
    )
    return fn(e2, ones16, zeros16)


NB = 4
NROUND = NCHA // NB


def _agg_body(th_hbm, snd_hbm, rcv_hbm, zeros_hbm, out_hbm,
              snd_v, rcv_v, rbs, gsems, ssems, acc):
    c = lax.axis_index("c")
    s = lax.axis_index("s")
    pltpu.sync_copy(snd_hbm.at[s], snd_v)
    pltpu.sync_copy(rcv_hbm.at[s], rcv_v)
    row = pl.ds(s * RPT, RPT)
    pltpu.sync_copy(zeros_hbm, acc.at[row])
    plsc.subcore_barrier()
    tc_ref = th_hbm.at[c]

    for b in range(NB):
        pltpu.async_copy(tc_ref.at[snd_v.at[b]], rbs.at[b], gsems.at[b])

    def body(jj, carry):
        base = jj * NB
        for b in range(NB):
            j = base + b
            pltpu.make_async_copy(tc_ref.at[snd_v.at[j]], rbs.at[b],
                                  gsems.at[b]).wait()
            pltpu.async_copy(rbs.at[b], acc.at[rcv_v.at[j]], ssems.at[b],
                             add=True)
        for b in range(NB):
            j = base + b
            pltpu.make_async_copy(rbs.at[b], acc.at[rcv_v.at[j]],
                                  ssems.at[b]).wait()

            @pl.when(jj + 1 < NROUND)
            def _():
                pltpu.async_copy(tc_ref.at[snd_v.at[j + NB]], rbs.at[b],
                                 gsems.at[b])

        return carry

    lax.fori_loop(0, NROUND, body, 0)
    plsc.subcore_barrier()
    pltpu.sync_copy(acc.at[row], out_hbm.at[c, row])


EB = 128


def _aggf_body(th_hbm, snd_hbm, rcv_hbm, zeros_hbm, nrb_hbm, out_hbm,
               snd_v, rcv_v, rbs, gsems, ssems, nrb_v, buf_v, acc):
    c = lax.axis_index("c")
    s = lax.axis_index("s")
    pltpu.sync_copy(snd_hbm.at[s], snd_v)
    pltpu.sync_copy(rcv_hbm.at[s], rcv_v)
    row = pl.ds(s * RPT, RPT)
    pltpu.sync_copy(zeros_hbm, acc.at[row])
    plsc.subcore_barrier()
    tc_ref = th_hbm.at[c]

    for b in range(NB):
        pltpu.async_copy(tc_ref.at[snd_v.at[b]], rbs.at[b], gsems.at[b])

    def body(jj, carry):
        base = jj * NB
        for b in range(NB):
            j = base + b
            pltpu.make_async_copy(tc_ref.at[snd_v.at[j]], rbs.at[b],
                                  gsems.at[b]).wait()
            pltpu.async_copy(rbs.at[b], acc.at[rcv_v.at[j]], ssems.at[b],
                             add=True)
        for b in range(NB):
            j = base + b
            pltpu.make_async_copy(rbs.at[b], acc.at[rcv_v.at[j]],
                                  ssems.at[b]).wait()

            @pl.when(jj + 1 < NROUND)
            def _():
                pltpu.async_copy(tc_ref.at[snd_v.at[j + NB]], rbs.at[b],
                                 gsems.at[b])

        return carry

    lax.fori_loop(0, NROUND, body, 0)
    plsc.subcore_barrier()
    def eblk(t, carry):
        r0 = s * RPT + t * EB
        pltpu.sync_copy(acc.at[pl.ds(r0, EB)], buf_v)
        pltpu.sync_copy(nrb_hbm.at[pl.ds(r0, EB)], nrb_v)

        def srow(r, carry2):
            nv = nrb_v[r]
            for k in range(DH // 16):
                col = pl.ds(k * 16, 16)
                buf_v[r, col] = jnp.maximum(buf_v[r, col] * nv, 0.0)
            return carry2

        lax.fori_loop(0, EB, srow, 0)
        pltpu.sync_copy(buf_v, out_hbm.at[pl.ds(r0, EB), pl.ds(c * DH, DH)])
        return carry

    lax.fori_loop(0, RPT // EB, eblk, 0)


def _aggf_call(th, snd, rcv, zerosH, nrb):
    fn = pl.kernel(
        _aggf_body,
        out_type=jax.ShapeDtypeStruct((NP, D), jnp.float32),
        mesh=plsc.VectorSubcoreMesh(**_MESH),
        compiler_params=pltpu.CompilerParams(use_tc_tiling_on_sc=False),
        scratch_types=[
            pltpu.VMEM((NCHA, CA), jnp.int32),
            pltpu.VMEM((NCHA, CA), jnp.int32),
            pltpu.VMEM((NB, CA, DH), jnp.float32),
            pltpu.SemaphoreType.DMA((NB,)),
            pltpu.SemaphoreType.DMA((NB,)),
            pltpu.VMEM((EB, DEG_L), jnp.float32),
            pltpu.VMEM((EB, DH), jnp.float32),
            pltpu.VMEM_SHARED((NP, DH), jnp.float32),
        ],
    )
    return fn(th, snd, rcv, zerosH, nrb)


def _agg_call(th, snd, rcv, zerosH):
    fn = pl.kernel(
        _agg_body,
        out_type=jax.ShapeDtypeStruct((NC, NP, DH), jnp.float32),
        mesh=plsc.VectorSubcoreMesh(**_MESH),
        compiler_params=pltpu.CompilerParams(use_tc_tiling_on_sc=False),
        scratch_types=[
            pltpu.VMEM((NCHA, CA), jnp.int32),
            pltpu.VMEM((NCHA, CA), jnp.int32),
            pltpu.VMEM((NB, CA, DH), jnp.float32),
            pltpu.SemaphoreType.DMA((NB,)),
            pltpu.SemaphoreType.DMA((NB,)),
            pltpu.VMEM_SHARED((NP, DH), jnp.float32),
        ],
    )
    return fn(th, snd, rcv, zerosH)


def _l1_body(x_ref, w_ref, b_ref, dp_ref, o_ref):
    ns = lax.rsqrt(jnp.maximum(dp_ref[0, :N], 1.0))
    h = jnp.dot(x_ref[...], w_ref[...], preferred_element_type=jnp.float32)
    h = (h + b_ref[...][None, :]) * ns[:, None]
    o_ref[0] = h[:, :DH]
    o_ref[1] = h[:, DH:]


def _l1_call(x, W1, b1, dp):
    return pl.pallas_call(
        _l1_body,
        out_shape=jax.ShapeDtypeStruct((NC, N, DH), jnp.float32),
    )(x, W1, b1, dp)


def _l2_body(p_ref, w_ref, b_ref, dp_ref, o_ref, onr_ref):
    nr_full = lax.rsqrt(jnp.maximum(dp_ref[1], 1.0))
    nr = nr_full[:N]
    ns = lax.rsqrt(jnp.maximum(dp_ref[0, :N], 1.0))
    a = jnp.concatenate([p_ref[0, :N], p_ref[1, :N]], axis=-1)
    h1 = jnp.maximum(a * nr[:, None], 0.0)
    h = jnp.dot(h1, w_ref[...], preferred_element_type=jnp.float32)
    h = (h + b_ref[...][None, :]) * ns[:, None]
    o_ref[0] = h[:, :DH]
    o_ref[1] = h[:, DH:]
    onr_ref[...] = jnp.broadcast_to(nr_full[:, None], (NP, DEG_L))


def _l2_call(p, W2, b2, dp):
    return pl.pallas_call(
        _l2_body,
        out_shape=[jax.ShapeDtypeStruct((NC, N, DH), jnp.float32),
                   jax.ShapeDtypeStruct((NP, DEG_L), jnp.float32)],
    )(p, W2, b2, dp)


def _out_body(p_ref, dp_ref, o_ref):
    nr = lax.rsqrt(jnp.maximum(dp_ref[1, :N], 1.0))
    a = jnp.concatenate([p_ref[0, :N], p_ref[1, :N]], axis=-1)
    o_ref[...] = jnp.maximum(a * nr[:, None], 0.0)


def _out_call(p, dp):
    return pl.pallas_call(
        _out_body,
        out_shape=jax.ShapeDtypeStruct((N, D), jnp.float32),
    )(p, dp)


def kernel(x, edge_index, W1, b1, W2, b2):
    e2 = edge_index.reshape(2, NS, NCHUNK, C)
    npad = EPA - E
    pad_s = (jnp.arange(npad, dtype=jnp.int32) * 61) % N
    pad_r = N + (jnp.arange(npad, dtype=jnp.int32) % (NP - N))
    snd = jnp.concatenate([edge_index[0], pad_s]).reshape(NS, NCHA, CA)
    rcv = jnp.concatenate([edge_index[1], pad_r]).reshape(NS, NCHA, CA)
    ones16 = jnp.ones((C, DEG_L), jnp.float32)
    zeros16 = jnp.zeros((RPT, DEG_L), jnp.float32)
    zerosH = jnp.zeros((RPT, DH), jnp.float32)

    degp = _deg_call(e2, ones16, zeros16)
    dp = degp[..., 0]
    t1 = _l1_call(x, W1, b1, dp)
    p1 = _agg_call(t1, snd, rcv, zerosH)
    t2, nrb = _l2_call(p1, W2, b2, dp)
    out = _aggf_call(t2, snd, rcv, zerosH, nrb)
    return out[:N]
